# baseline (speedup 1.0000x reference)
import jax
import jax.numpy as jnp
from jax.experimental import pallas as pl
from jax.experimental.pallas import tpu as pltpu

_N = 16
_C = 255


def _act(x):
    c = jax.lax.broadcasted_iota(jnp.int32, x.shape, 0) % 85
    raw = (c == 2) | (c == 3)
    return jnp.where(raw, x, jax.nn.sigmoid(x))


def _body(a_ref, b_ref, c_ref, o_ref):
    i = pl.program_id(1)

    for k in range(10):
        @pl.when(i == k)
        def _(k=k):
            x = a_ref[0].reshape(_C, 608)
            o_ref[0, 608 * k:608 * (k + 1), :] = _act(x).T

    @pl.when(i == 10)
    def _():
        x = b_ref[0].reshape(_C, 1444)
        o_ref[0, 5776:7220, :] = _act(x).T

    @pl.when(i == 11)
    def _():
        x = c_ref[0].reshape(_C, 361)
        o_ref[0, 7220:7581, :] = _act(x).T


def kernel(fm0, fm1, fm2, cell_anchors):
    del cell_anchors
    out = pl.pallas_call(
        _body,
        grid=(_N, 12),
        in_specs=[
            pl.BlockSpec((1, _C, 8, 76), lambda n, i: (n, 0, jnp.minimum(i, 9), 0)),
            pl.BlockSpec((1, _C, 38, 38), lambda n, i: (n, 0, 0, 0)),
            pl.BlockSpec((1, _C, 19, 19), lambda n, i: (n, 0, 0, 0)),
        ],
        out_specs=pl.BlockSpec((1, 7581, _C), lambda n, i: (n, 0, 0)),
        out_shape=jax.ShapeDtypeStruct((_N, 7581, _C), jnp.float32),
    )(fm0, fm1, fm2)
    return out.reshape(_N, 7581, 3, 85)


# speedup vs baseline: 2.7965x; 2.7965x over previous
import jax
import jax.numpy as jnp
from jax.experimental import pallas as pl
from jax.experimental.pallas import tpu as pltpu

_N = 16
_C = 255
_R = 304


def _act(x):
    c = jax.lax.broadcasted_iota(jnp.int32, x.shape, x.ndim - 1) % 85
    raw = (c == 2) | (c == 3)
    return jnp.where(raw, x, jax.nn.sigmoid(x))


def _tr(x):
    return _act(jnp.transpose(x, (1, 0, 2)))


def _body(a_ref, b_ref, c_ref, o_ref):
    i = pl.program_id(0)

    @pl.when(i < 19)
    def _():
        o_ref[...] = _tr(a_ref[...])

    @pl.when((i >= 19) & (i < 23))
    def _():
        o_ref[...] = _tr(b_ref[...])

    @pl.when(i == 23)
    def _():
        y1 = _tr(b_ref[...])
        y2 = _tr(c_ref[...])
        o_ref[:, 0:228, :] = y1[:, 0:228, :]
        o_ref[:, 228:304, :] = y2[:, 0:76, :]

    @pl.when(i == 24)
    def _():
        y2 = _tr(c_ref[...])
        o_ref[:, 0:285, :] = y2[:, 76:361, :]


def kernel(fm0, fm1, fm2, cell_anchors):
    del cell_anchors
    at = jnp.transpose(fm0, (2, 3, 0, 1)).reshape(5776, _N, _C)
    bt = jnp.transpose(fm1, (2, 3, 0, 1)).reshape(1444, _N, _C)
    ct = jnp.transpose(fm2, (2, 3, 0, 1)).reshape(361, _N, _C)
    out = pl.pallas_call(
        _body,
        grid=(25,),
        in_specs=[
            pl.BlockSpec((_R, _N, _C), lambda i: (jnp.minimum(i, 18), 0, 0)),
            pl.BlockSpec((_R, _N, _C), lambda i: (jnp.clip(i - 19, 0, 4), 0, 0)),
            pl.BlockSpec((361, _N, _C), lambda i: (0, 0, 0)),
        ],
        out_specs=pl.BlockSpec((_N, _R, _C), lambda i: (0, i, 0)),
        out_shape=jax.ShapeDtypeStruct((_N, 7581, _C), jnp.float32),
    )(at, bt, ct)
    return out.reshape(_N, 7581, 3, 85)


# speedup vs baseline: 4.8868x; 1.7474x over previous
import jax
import jax.numpy as jnp
from jax.experimental import pallas as pl
from jax.experimental.pallas import tpu as pltpu

_N = 16
_C = 255
_W = 256


def _act(x):
    c = jax.lax.broadcasted_iota(jnp.int32, x.shape, x.ndim - 1) % 85
    raw = (c == 2) | (c == 3)
    return jnp.where(raw, x, jax.nn.sigmoid(x))


def _body(a_ref, blo_ref, bhi_ref, c_ref, o_ref):
    k = pl.program_id(0)

    def emit(x):
        u = jnp.transpose(_act(x), (1, 0, 2))
        v = jnp.transpose(u, (0, 2, 1))
        o_ref[...] = jnp.transpose(v, (1, 0, 2))

    @pl.when(k <= 21)
    def _():
        emit(a_ref[...])

    @pl.when(k == 22)
    def _():
        emit(jnp.concatenate([a_ref[0:144], blo_ref[0:112]], axis=0))

    @pl.when((k >= 23) & (k <= 27))
    def _():
        emit(jnp.concatenate([blo_ref[112:256], bhi_ref[0:112]], axis=0))

    @pl.when(k == 28)
    def _():
        emit(jnp.concatenate([blo_ref[112:164], c_ref[0:204]], axis=0))

    @pl.when(k == 29)
    def _():
        emit(jnp.concatenate([c_ref[204:361], c_ref[0:99]], axis=0))


def kernel(fm0, fm1, fm2, cell_anchors):
    del cell_anchors
    at = jnp.transpose(fm0, (2, 3, 0, 1)).reshape(5776, _N, _C)
    bt = jnp.transpose(fm1, (2, 3, 0, 1)).reshape(1444, _N, _C)
    ct = jnp.transpose(fm2, (2, 3, 0, 1)).reshape(361, _N, _C)
    out = pl.pallas_call(
        _body,
        grid=(30,),
        in_specs=[
            pl.BlockSpec((_W, _N, _C), lambda k: (jnp.minimum(k, 22), 0, 0)),
            pl.BlockSpec((_W, _N, _C), lambda k: (jnp.clip(k - 23, 0, 5), 0, 0)),
            pl.BlockSpec((_W, _N, _C), lambda k: (jnp.clip(k - 22, 0, 5), 0, 0)),
            pl.BlockSpec((361, _N, _C), lambda k: (0, 0, 0)),
        ],
        out_specs=pl.BlockSpec((_C, _N, _W), lambda k: (0, 0, k)),
        out_shape=jax.ShapeDtypeStruct((_C, _N, 7581), jnp.float32),
    )(at, bt, bt, ct)
    return jnp.transpose(out.reshape(3, 85, _N, 7581), (2, 3, 0, 1))


# speedup vs baseline: 5.0508x; 1.0336x over previous
import jax
import jax.numpy as jnp
from jax.experimental import pallas as pl
from jax.experimental.pallas import tpu as pltpu

_N = 16
_C = 255
_W = 256


def _act(x):
    c = jax.lax.broadcasted_iota(jnp.int32, x.shape, x.ndim - 1) % 85
    raw = (c == 2) | (c == 3)
    return jnp.where(raw, x, jax.nn.sigmoid(x))


def _body(a_ref, blo_ref, bhi_ref, c_ref, o_ref):
    k = pl.program_id(0)

    def emit(x):
        u = jnp.transpose(x, (1, 0, 2))
        v = jnp.transpose(u, (0, 2, 1))
        t = jnp.transpose(v, (1, 0, 2))
        o_ref[...] = jax.nn.sigmoid(t)
        for ch in (2, 3, 87, 88, 172, 173):
            o_ref[ch] = t[ch]

    @pl.when(k <= 21)
    def _():
        emit(a_ref[...])

    @pl.when(k == 22)
    def _():
        emit(jnp.concatenate([a_ref[0:144], blo_ref[0:112]], axis=0))

    @pl.when((k >= 23) & (k <= 27))
    def _():
        emit(jnp.concatenate([blo_ref[112:256], bhi_ref[0:112]], axis=0))

    @pl.when(k == 28)
    def _():
        emit(jnp.concatenate([blo_ref[112:164], c_ref[0:204]], axis=0))

    @pl.when(k == 29)
    def _():
        emit(jnp.concatenate([c_ref[204:361], c_ref[0:99]], axis=0))


def kernel(fm0, fm1, fm2, cell_anchors):
    del cell_anchors
    at = jnp.transpose(fm0, (2, 3, 0, 1)).reshape(5776, _N, _C)
    bt = jnp.transpose(fm1, (2, 3, 0, 1)).reshape(1444, _N, _C)
    ct = jnp.transpose(fm2, (2, 3, 0, 1)).reshape(361, _N, _C)
    out = pl.pallas_call(
        _body,
        grid=(30,),
        in_specs=[
            pl.BlockSpec((_W, _N, _C), lambda k: (jnp.minimum(k, 22), 0, 0)),
            pl.BlockSpec((_W, _N, _C), lambda k: (jnp.clip(k - 23, 0, 5), 0, 0)),
            pl.BlockSpec((_W, _N, _C), lambda k: (jnp.clip(k - 22, 0, 5), 0, 0)),
            pl.BlockSpec((361, _N, _C), lambda k: (0, 0, 0)),
        ],
        out_specs=pl.BlockSpec((_C, _N, _W), lambda k: (0, 0, k)),
        out_shape=jax.ShapeDtypeStruct((_C, _N, 7581), jnp.float32),
    )(at, bt, bt, ct)
    return jnp.transpose(out.reshape(3, 85, _N, 7581), (2, 3, 0, 1))


# speedup vs baseline: 5.1097x; 1.0117x over previous
import jax
import jax.numpy as jnp
from jax.experimental import pallas as pl
from jax.experimental.pallas import tpu as pltpu

_N = 16
_C = 255
_W = 256


def _act(x):
    c = jax.lax.broadcasted_iota(jnp.int32, x.shape, x.ndim - 1) % 85
    raw = (c == 2) | (c == 3)
    return jnp.where(raw, x, jax.nn.sigmoid(x))


def _body(a_ref, blo_ref, bhi_ref, c_ref, o_ref):
    k = pl.program_id(0)

    def emit(x):
        u = jnp.transpose(x, (1, 0, 2))
        v = jnp.transpose(u, (0, 2, 1))
        t = jnp.transpose(v, (1, 0, 2))
        o_ref[...] = 0.5 * jnp.tanh(0.5 * t) + 0.5
        for ch in (2, 3, 87, 88, 172, 173):
            o_ref[ch] = t[ch]

    @pl.when(k <= 21)
    def _():
        emit(a_ref[...])

    @pl.when(k == 22)
    def _():
        emit(jnp.concatenate([a_ref[0:144], blo_ref[0:112]], axis=0))

    @pl.when((k >= 23) & (k <= 27))
    def _():
        emit(jnp.concatenate([blo_ref[112:256], bhi_ref[0:112]], axis=0))

    @pl.when(k == 28)
    def _():
        emit(jnp.concatenate([blo_ref[112:164], c_ref[0:204]], axis=0))

    @pl.when(k == 29)
    def _():
        emit(jnp.concatenate([c_ref[204:361], c_ref[0:99]], axis=0))


def kernel(fm0, fm1, fm2, cell_anchors):
    del cell_anchors
    at = jnp.transpose(fm0, (2, 3, 0, 1)).reshape(5776, _N, _C)
    bt = jnp.transpose(fm1, (2, 3, 0, 1)).reshape(1444, _N, _C)
    ct = jnp.transpose(fm2, (2, 3, 0, 1)).reshape(361, _N, _C)
    out = pl.pallas_call(
        _body,
        grid=(30,),
        in_specs=[
            pl.BlockSpec((_W, _N, _C), lambda k: (jnp.minimum(k, 22), 0, 0)),
            pl.BlockSpec((_W, _N, _C), lambda k: (jnp.clip(k - 23, 0, 5), 0, 0)),
            pl.BlockSpec((_W, _N, _C), lambda k: (jnp.clip(k - 22, 0, 5), 0, 0)),
            pl.BlockSpec((361, _N, _C), lambda k: (0, 0, 0)),
        ],
        out_specs=pl.BlockSpec((_C, _N, _W), lambda k: (0, 0, k)),
        out_shape=jax.ShapeDtypeStruct((_C, _N, 7581), jnp.float32),
    )(at, bt, bt, ct)
    return jnp.transpose(out.reshape(3, 85, _N, 7581), (2, 3, 0, 1))
